# Initial kernel scaffold; baseline (speedup 1.0000x reference)
import functools

import jax
import jax.numpy as jnp
from jax import lax
from jax.experimental import pallas as pl
from jax.experimental.pallas import tpu as pltpu
from jax.experimental.pallas import tpu_sc as plsc

VOCAB = 100000
D = 128
B = 1024
S = 512
N = B * S

_info = plsc.get_sparse_core_info()
NC = _info.num_cores
NS = _info.num_subcores
NW = NC * NS
B_PER_W = N // NW
CHUNK = 512
NCHUNK = B_PER_W // CHUNK

_mesh = plsc.VectorSubcoreMesh(core_axis_name="c", subcore_axis_name="s")


@functools.partial(
    pl.kernel,
    mesh=_mesh,
    out_type=jax.ShapeDtypeStruct((N, D), jnp.float32),
    scratch_types=[
        pltpu.VMEM((CHUNK,), jnp.int32),
        pltpu.VMEM((CHUNK, D), jnp.float32),
        pltpu.SemaphoreType.DMA,
    ],
)
def _sc_gather(table_hbm, idx_hbm, out_hbm, idx_v, rows_v, sem):
    wid = lax.axis_index("s") * NC + lax.axis_index("c")
    base = wid * B_PER_W

    def body(i, carry):
        off = base + i * CHUNK
        pltpu.sync_copy(idx_hbm.at[pl.ds(off, CHUNK)], idx_v)
        pltpu.async_copy(table_hbm.at[idx_v], rows_v, sem).wait()
        pltpu.sync_copy(rows_v, out_hbm.at[pl.ds(off, CHUNK)])
        return carry

    lax.fori_loop(0, NCHUNK, body, 0)


_BB = 8


def _tc_ln_body(g_ref, seg_ref, segt_ref, pos_ref, w_ref, b_ref, o_ref):
    x = g_ref[...]
    seg = seg_ref[...]
    s0 = segt_ref[0, :]
    s1 = segt_ref[1, :]
    seg_emb = jnp.where((seg == 1)[..., None], s1[None, None, :],
                        s0[None, None, :])
    x = x + seg_emb + pos_ref[...][None, :, :]
    mean = jnp.mean(x, axis=-1, keepdims=True)
    var = jnp.mean(jnp.square(x - mean), axis=-1, keepdims=True)
    xh = (x - mean) * lax.rsqrt(var + 1e-5)
    o_ref[...] = xh * w_ref[...] + b_ref[...]


def _tc_ln(gathered, segment_ids, segment_table, position_table, w, b):
    grid = (B // _BB,)
    return pl.pallas_call(
        _tc_ln_body,
        grid=grid,
        in_specs=[
            pl.BlockSpec((_BB, S, D), lambda i: (i, 0, 0)),
            pl.BlockSpec((_BB, S), lambda i: (i, 0)),
            pl.BlockSpec((2, D), lambda i: (0, 0)),
            pl.BlockSpec((S, D), lambda i: (0, 0)),
            pl.BlockSpec((D,), lambda i: (0,)),
            pl.BlockSpec((D,), lambda i: (0,)),
        ],
        out_specs=pl.BlockSpec((_BB, S, D), lambda i: (i, 0, 0)),
        out_shape=jax.ShapeDtypeStruct((B, S, D), jnp.float32),
    )(gathered, segment_ids, segment_table, position_table, w, b)


def kernel(token_ids, segment_ids, token_table, segment_table, position_table,
           ln_weight, ln_bias):
    flat_ids = token_ids.reshape(N).astype(jnp.int32)
    gathered = _sc_gather(token_table, flat_ids)
    gathered = gathered.reshape(B, S, D)
    return _tc_ln(gathered, segment_ids, segment_table, position_table,
                  ln_weight, ln_bias)


# speedup vs baseline: 6.4365x; 6.4365x over previous
import functools

import jax
import jax.numpy as jnp
from jax import lax
from jax.experimental import pallas as pl
from jax.experimental.pallas import tpu as pltpu
from jax.experimental.pallas import tpu_sc as plsc

VOCAB = 100000
D = 128
B = 1024
S = 512
N = B * S

_info = plsc.get_sparse_core_info()
NC = _info.num_cores
NS = _info.num_subcores
NW = NC * NS
B_PER_W = N // NW
CHUNK = 512
NCHUNK = B_PER_W // CHUNK

_mesh = plsc.VectorSubcoreMesh(core_axis_name="c", subcore_axis_name="s")


@functools.partial(
    pl.kernel,
    mesh=_mesh,
    out_type=jax.ShapeDtypeStruct((N, D), jnp.float32),
    scratch_types=[
        pltpu.VMEM((CHUNK,), jnp.int32),
        pltpu.VMEM((CHUNK, D), jnp.float32),
        pltpu.SemaphoreType.DMA,
    ],
)
def _sc_gather(table_hbm, idx_hbm, out_hbm, idx_v, rows_v, sem):
    wid = lax.axis_index("s") * NC + lax.axis_index("c")
    base = wid * B_PER_W

    def body(i, carry):
        off = base + i * CHUNK
        pltpu.sync_copy(idx_hbm.at[pl.ds(off, CHUNK)], idx_v)
        pltpu.async_copy(table_hbm.at[idx_v], rows_v, sem).wait()
        pltpu.sync_copy(rows_v, out_hbm.at[pl.ds(off, CHUNK)])
        return carry

    lax.fori_loop(0, NCHUNK, body, 0)


_BB = 8


def _tc_ln_body(g_ref, seg_ref, segt_ref, pos_ref, w_ref, b_ref, o_ref):
    x = g_ref[...]
    segf = seg_ref[...]
    s0 = segt_ref[0, :]
    s1 = segt_ref[1, :]
    base = pos_ref[...] + s0[None, :]
    x = x + base[None, :, :] + segf * (s1 - s0)[None, None, :]
    mean = jnp.mean(x, axis=-1, keepdims=True)
    var = jnp.mean(jnp.square(x - mean), axis=-1, keepdims=True)
    xh = (x - mean) * lax.rsqrt(var + 1e-5)
    o_ref[...] = xh * w_ref[...] + b_ref[...]


def _tc_ln(gathered, segment_ids, segment_table, position_table, w, b):
    grid = (B // _BB,)
    return pl.pallas_call(
        _tc_ln_body,
        grid=grid,
        in_specs=[
            pl.BlockSpec((_BB, S, D), lambda i: (i, 0, 0)),
            pl.BlockSpec((_BB, S, 1), lambda i: (i, 0, 0)),
            pl.BlockSpec((2, D), lambda i: (0, 0)),
            pl.BlockSpec((S, D), lambda i: (0, 0)),
            pl.BlockSpec((D,), lambda i: (0,)),
            pl.BlockSpec((D,), lambda i: (0,)),
        ],
        out_specs=pl.BlockSpec((_BB, S, D), lambda i: (i, 0, 0)),
        out_shape=jax.ShapeDtypeStruct((B, S, D), jnp.float32),
    )(gathered, segment_ids, segment_table, position_table, w, b)


def kernel(token_ids, segment_ids, token_table, segment_table, position_table,
           ln_weight, ln_bias):
    flat_ids = token_ids.reshape(N).astype(jnp.int32)
    seg3 = segment_ids.astype(jnp.float32).reshape(B, S, 1)
    gathered = _sc_gather(token_table, flat_ids)
    gathered = gathered.reshape(B, S, D)
    return _tc_ln(gathered, seg3, segment_table, position_table,
                  ln_weight, ln_bias)
